# Initial kernel scaffold; baseline (speedup 1.0000x reference)
import jax
import jax.numpy as jnp
from jax.experimental import pallas as pl


def kernel(x, edge_index, params):
    raise NotImplementedError("write your pallas kernel here")


# speedup vs baseline: 1.5626x; 1.5626x over previous
import functools

import jax
import jax.numpy as jnp
from jax import lax
from jax.experimental import pallas as pl
from jax.experimental.pallas import tpu as pltpu
from jax.experimental.pallas import tpu_sc as plsc

NC, NS = 2, 16
NW = NC * NS
KE = 128
BR = 400


def _sc_agg_call(table, srcp, dstp, n_nodes, fc, w):
    ep = srcp.shape[0]
    ept = ep // NW
    nch = ept // KE
    npad = n_nodes + 16
    rz = (n_nodes // NS) // 8 * 8
    rrem = npad - NS * rz
    orem = n_nodes - NS * rz
    mesh = plsc.VectorSubcoreMesh(core_axis_name="c", subcore_axis_name="s")

    @functools.partial(
        pl.kernel,
        out_type=jax.ShapeDtypeStruct((NC, fc, n_nodes, w), jnp.float32),
        mesh=mesh,
        scratch_types=[
            pltpu.VMEM((KE,), jnp.int32),
            pltpu.VMEM((KE,), jnp.int32),
            pltpu.VMEM((KE, w), jnp.float32),
            pltpu.VMEM((48, w), jnp.float32),
            pltpu.VMEM_SHARED((npad, w), jnp.float32),
            pltpu.SemaphoreType.DMA,
        ],
    )
    def k(table_r, src_r, dst_r, out_r, src_v, dst_v, rows_v, zbuf, acc, sem):
        cid = lax.axis_index("c")
        sid = lax.axis_index("s")
        wid = sid * NC + cid
        zv = jnp.zeros((16,), jnp.float32)

        def zrow(i, _):
            def zcol(j, _):
                zbuf[i, pl.ds(j * 16, 16)] = zv
                return 0
            return lax.fori_loop(0, w // 16, zcol, 0)
        lax.fori_loop(0, 48, zrow, 0)

        ebase = wid * ept
        for c in range(fc):
            def zcp(t, _):
                pltpu.sync_copy(zbuf, acc.at[pl.ds(sid * rz + t * 48, 48)])
                return 0
            lax.fori_loop(0, rz // 48, zcp, 0)

            @pl.when(sid == 0)
            def _():
                pltpu.sync_copy(zbuf.at[pl.ds(0, rrem)],
                                acc.at[pl.ds(NS * rz, rrem)])
            plsc.subcore_barrier()

            def chunk(ci, _):
                off = ebase + ci * KE
                pltpu.sync_copy(src_r.at[pl.ds(off, KE)], src_v)
                pltpu.sync_copy(dst_r.at[pl.ds(off, KE)], dst_v)
                if fc > 1:
                    def scale(j, _):
                        sl = pl.ds(j * 16, 16)
                        src_v[sl] = src_v[sl] * fc + c
                        return 0
                    lax.fori_loop(0, KE // 16, scale, 0)
                pltpu.async_copy(table_r.at[src_v], rows_v, sem).wait()
                pltpu.sync_copy(rows_v, acc.at[dst_v], add=True)
                return 0
            lax.fori_loop(0, nch, chunk, 0)
            plsc.subcore_barrier()
            pltpu.sync_copy(acc.at[pl.ds(sid * rz, rz)],
                            out_r.at[cid, c, pl.ds(sid * rz, rz)])

            @pl.when(sid == 0)
            def _():
                pltpu.sync_copy(acc.at[pl.ds(NS * rz, orem)],
                                out_r.at[cid, c, pl.ds(NS * rz, orem)])
            plsc.subcore_barrier()

    return k(table, srcp, dstp)


def _agg(xfeat, srcp, dstp):
    n, f = xfeat.shape
    if f <= 128:
        return _sc_agg_call(xfeat, srcp, dstp, n, 1, f)
    fc = f // 128
    return _sc_agg_call(xfeat.reshape(n * fc, 128), srcp, dstp, n, fc, 128)


def _layer1_call(x0, p, w1p, b1, n):
    gi = n // BR

    def body(x_ref, p_ref, w_ref, b_ref, h_ref, inv_ref):
        ps = p_ref[0, 0] + p_ref[1, 0]
        degc = jnp.maximum(ps[:, 1:2], 1.0)
        a = x_ref[...] + ps / degc
        h = jnp.dot(a, w_ref[...], preferred_element_type=jnp.float32, precision=lax.Precision.HIGHEST)
        h = jnp.maximum(h + b_ref[...], 0.0)
        h_ref[...] = jnp.concatenate([h, jnp.zeros((BR, 64), jnp.float32)], axis=1)
        inv_ref[...] = jnp.broadcast_to(degc, (BR, 128))

    return pl.pallas_call(
        body,
        grid=(gi,),
        in_specs=[
            pl.BlockSpec((BR, 128), lambda i: (i, 0)),
            pl.BlockSpec((2, 1, BR, 128), lambda i: (0, 0, i, 0)),
            pl.BlockSpec((128, 64), lambda i: (0, 0)),
            pl.BlockSpec((1, 64), lambda i: (0, 0)),
        ],
        out_specs=[
            pl.BlockSpec((BR, 128), lambda i: (i, 0)),
            pl.BlockSpec((BR, 128), lambda i: (i, 0)),
        ],
        out_shape=[
            jax.ShapeDtypeStruct((n, 128), jnp.float32),
            jax.ShapeDtypeStruct((n, 128), jnp.float32),
        ],
    )(x0, p, w1p, b1)


def _mpl_dense(xc, p, invd, w, b, concat, xw=None, want_stats=False):
    n, f = xc.shape
    cdim, wch = p.shape[1], p.shape[3]
    fo = w.shape[1]
    xw = f if xw is None else xw
    ftot = (xw if concat else 0) + fo
    gi = n // BR

    def body(*refs):
        if concat:
            xc_ref, p_ref, inv_ref, w_ref, b_ref, xf_ref = refs[:6]
            rest = refs[6:]
        else:
            xc_ref, p_ref, inv_ref, w_ref, b_ref = refs[:5]
            rest = refs[5:]
        if want_stats:
            out_ref, st_ref, acc_ref = rest
        else:
            out_ref, acc_ref = rest
        c = pl.program_id(1)
        ps = p_ref[0, 0] + p_ref[1, 0]
        a = xc_ref[...] + ps / inv_ref[:, :wch]
        pp = jnp.dot(a, w_ref[...], preferred_element_type=jnp.float32, precision=lax.Precision.HIGHEST)

        @pl.when(c == 0)
        def _():
            acc_ref[...] = pp

        @pl.when(c > 0)
        def _():
            acc_ref[...] += pp

        @pl.when(c == cdim - 1)
        def _():
            res = jnp.maximum(acc_ref[...] + b_ref[...], 0.0)
            full = jnp.concatenate([xf_ref[:, :xw], res], axis=1) if concat else res
            out_ref[...] = full
            if want_stats:
                s1 = jnp.sum(full, axis=0, keepdims=True)[None]
                s2 = jnp.sum(full * full, axis=0, keepdims=True)[None]
                st_ref[...] = jnp.concatenate([s1, s2], axis=1)

    in_specs = [
        pl.BlockSpec((BR, wch), (lambda i, c: (i, c)) if cdim > 1 else (lambda i, c: (i, 0))),
        pl.BlockSpec((2, 1, BR, wch), lambda i, c: (0, c, i, 0)),
        pl.BlockSpec((BR, 128), lambda i, c: (i, 0)),
        pl.BlockSpec((wch, fo), lambda i, c: (c, 0)),
        pl.BlockSpec((1, fo), lambda i, c: (0, 0)),
    ]
    args = [xc, p, invd, w, b.reshape(1, fo)]
    if concat:
        in_specs.append(pl.BlockSpec((BR, f), lambda i, c: (i, 0)))
        args.append(xc)
    out_specs = [pl.BlockSpec((BR, ftot), lambda i, c: (i, 0))]
    out_shape = [jax.ShapeDtypeStruct((n, ftot), jnp.float32)]
    if want_stats:
        out_specs.append(pl.BlockSpec((1, 2, ftot), lambda i, c: (i, 0, 0)))
        out_shape.append(jax.ShapeDtypeStruct((gi, 2, ftot), jnp.float32))

    res = pl.pallas_call(
        body,
        grid=(gi, cdim),
        in_specs=in_specs,
        out_specs=out_specs,
        out_shape=out_shape,
        scratch_shapes=[pltpu.VMEM((BR, fo), jnp.float32)],
        compiler_params=pltpu.CompilerParams(
            dimension_semantics=("parallel", "arbitrary")),
    )(*args)
    return res if want_stats else res[0]


def _linear_multi(parts, ws, b, want_stats=True):
    n = parts[0].shape[0]
    fo = ws[0].shape[1]
    gi = n // BR
    npart = len(parts)

    def body(*refs):
        xrs = refs[:npart]
        wrs = refs[npart:2 * npart]
        b_ref = refs[2 * npart]
        if want_stats:
            out_ref, st_ref = refs[2 * npart + 1:]
        else:
            out_ref = refs[2 * npart + 1]
        acc = b_ref[...].astype(jnp.float32) * jnp.ones((BR, 1), jnp.float32)
        for xr, wr in zip(xrs, wrs):
            acc = acc + jnp.dot(xr[...], wr[...], preferred_element_type=jnp.float32, precision=lax.Precision.HIGHEST)
        out_ref[...] = acc
        if want_stats:
            s1 = jnp.sum(acc, axis=0, keepdims=True)[None]
            s2 = jnp.sum(acc * acc, axis=0, keepdims=True)[None]
            st_ref[...] = jnp.concatenate([s1, s2], axis=1)

    in_specs = [pl.BlockSpec((BR, pt.shape[1]), lambda i, _f=pt.shape[1]: (i, 0))
                for pt in parts]
    in_specs += [pl.BlockSpec(wv.shape, lambda i: (0, 0)) for wv in ws]
    in_specs.append(pl.BlockSpec((1, fo), lambda i: (0, 0)))
    out_specs = [pl.BlockSpec((BR, fo), lambda i: (i, 0))]
    out_shape = [jax.ShapeDtypeStruct((n, fo), jnp.float32)]
    if want_stats:
        out_specs.append(pl.BlockSpec((1, 2, fo), lambda i: (i, 0, 0)))
        out_shape.append(jax.ShapeDtypeStruct((gi, 2, fo), jnp.float32))
    return pl.pallas_call(
        body,
        grid=(gi,),
        in_specs=in_specs,
        out_specs=out_specs,
        out_shape=out_shape,
        compiler_params=pltpu.CompilerParams(dimension_semantics=("parallel",)),
    )(*parts, *ws, b.reshape(1, fo))


def _bn_mu(stats, n):
    gi, _, f = stats.shape

    def body(st_ref, out_ref):
        out_ref[...] = jnp.sum(st_ref[:, 0], axis=0, keepdims=True) / n

    return pl.pallas_call(
        body,
        grid=(1,),
        in_specs=[pl.BlockSpec((gi, 2, f), lambda i: (0, 0, 0))],
        out_specs=pl.BlockSpec((1, f), lambda i: (0, 0)),
        out_shape=jax.ShapeDtypeStruct((1, f), jnp.float32),
    )(stats)


def _bn_var_partial(y, mu):
    n, f = y.shape
    gi = n // BR

    def body(y_ref, mu_ref, out_ref):
        d = y_ref[...] - mu_ref[...]
        out_ref[...] = jnp.sum(d * d, axis=0, keepdims=True)[None]

    return pl.pallas_call(
        body,
        grid=(gi,),
        in_specs=[
            pl.BlockSpec((BR, f), lambda i: (i, 0)),
            pl.BlockSpec((1, f), lambda i: (0, 0)),
        ],
        out_specs=pl.BlockSpec((1, 1, f), lambda i: (i, 0, 0)),
        out_shape=jax.ShapeDtypeStruct((gi, 1, f), jnp.float32),
        compiler_params=pltpu.CompilerParams(dimension_semantics=("parallel",)),
    )(y, mu)


def _bn_ms(vstats, mu, n):
    gi, _, f = vstats.shape

    def body(v_ref, mu_ref, out_ref):
        var = jnp.sum(v_ref[:, 0], axis=0, keepdims=True) / n
        out_ref[...] = jnp.concatenate([mu_ref[...], jnp.sqrt(var + 1e-5)], axis=0)

    return pl.pallas_call(
        body,
        grid=(1,),
        in_specs=[
            pl.BlockSpec((gi, 1, f), lambda i: (0, 0, 0)),
            pl.BlockSpec((1, f), lambda i: (0, 0)),
        ],
        out_specs=pl.BlockSpec((2, f), lambda i: (0, 0)),
        out_shape=jax.ShapeDtypeStruct((2, f), jnp.float32),
    )(vstats, mu)


def _bn_apply(y, ms, g, be):
    n, f = y.shape
    gi = n // BR

    def body(y_ref, ms_ref, g_ref, be_ref, out_ref):
        out_ref[...] = (y_ref[...] - ms_ref[0:1]) / ms_ref[1:2] * g_ref[...] + be_ref[...]

    return pl.pallas_call(
        body,
        grid=(gi,),
        in_specs=[
            pl.BlockSpec((BR, f), lambda i: (i, 0)),
            pl.BlockSpec((2, f), lambda i: (0, 0)),
            pl.BlockSpec((1, f), lambda i: (0, 0)),
            pl.BlockSpec((1, f), lambda i: (0, 0)),
        ],
        out_specs=pl.BlockSpec((BR, f), lambda i: (i, 0)),
        out_shape=jax.ShapeDtypeStruct((n, f), jnp.float32),
        compiler_params=pltpu.CompilerParams(dimension_semantics=("parallel",)),
    )(y, ms, g.reshape(1, f), be.reshape(1, f))


def _bn(y, stats, g, be, n):
    mu = _bn_mu(stats, n)
    vst = _bn_var_partial(y, mu)
    ms = _bn_ms(vst, mu, n)
    return _bn_apply(y, ms, g, be)


def _ffp(parts, w1s, b1, w2, b2):
    n = parts[0].shape[0]
    gi = n // BR
    npart = len(parts)

    def body(*refs):
        xrs = refs[:npart]
        w1rs = refs[npart:2 * npart]
        b1_ref, w2_ref, b2_ref, out_ref = refs[2 * npart:]
        acc = b1_ref[...].astype(jnp.float32) * jnp.ones((BR, 1), jnp.float32)
        for xr, wr in zip(xrs, w1rs):
            acc = acc + jnp.dot(xr[...], wr[...], preferred_element_type=jnp.float32, precision=lax.Precision.HIGHEST)
        h = jnp.maximum(acc, 0.0)
        out_ref[...] = jnp.dot(h, w2_ref[...], preferred_element_type=jnp.float32, precision=lax.Precision.HIGHEST) + b2_ref[...]

    in_specs = [pl.BlockSpec((BR, pt.shape[1]), lambda i: (i, 0)) for pt in parts]
    in_specs += [pl.BlockSpec(wv.shape, lambda i: (0, 0)) for wv in w1s]
    in_specs += [
        pl.BlockSpec((1, 128), lambda i: (0, 0)),
        pl.BlockSpec((128, 1), lambda i: (0, 0)),
        pl.BlockSpec((1, 1), lambda i: (0, 0)),
    ]
    return pl.pallas_call(
        body,
        grid=(gi,),
        in_specs=in_specs,
        out_specs=pl.BlockSpec((BR, 1), lambda i: (i, 0)),
        out_shape=jax.ShapeDtypeStruct((n, 1), jnp.float32),
        compiler_params=pltpu.CompilerParams(dimension_semantics=("parallel",)),
    )(*parts, *w1s, b1.reshape(1, 128), w2, b2.reshape(1, 1))


def _xla_mpl(x, srcp, dstp, W, b, deg):
    n, f = x.shape
    fpad = ((f + 127) // 128) * 128
    xp = jnp.pad(x, ((0, 0), (0, fpad - f)))
    pt = _agg(xp, srcp, dstp)
    psum = (pt[0] + pt[1]).transpose(1, 0, 2).reshape(n, fpad)[:, :f]
    agg = psum / jnp.clip(deg, 1.0)[:, None]
    return jax.nn.relu((x + agg) @ W + b)


def kernel(x, edge_index, params):
    p = params
    n = x.shape[0]
    e = edge_index.shape[1]
    ep = ((e + NW * KE - 1) // (NW * KE)) * NW * KE
    src = jnp.asarray(edge_index[0], jnp.int32)
    dst = jnp.asarray(edge_index[1], jnp.int32)
    srcp = jnp.concatenate([src, jnp.zeros((ep - e,), jnp.int32)])
    dstp = jnp.concatenate([dst, jnp.full((ep - e,), n, jnp.int32)])
    deg = jax.ops.segment_sum(jnp.ones((e,), jnp.float32), dst, num_segments=n)

    def mpl(xx, W, b):
        return _xla_mpl(xx, srcp, dstp, W, b, deg)

    def green(xx, W, b):
        return jnp.concatenate([xx, mpl(xx, W, b)], axis=1)

    def bn(xx, g, be):
        mu = jnp.mean(xx, axis=0)
        var = jnp.var(xx, axis=0)
        return (xx - mu) / jnp.sqrt(var + 1e-5) * g + be

    s1 = mpl(x, p['W1'], p['b1'])
    s2 = green(s1, p['Wg1'], p['bg1'])
    s2 = green(s2, p['Wg2'], p['bg2'])
    s2 = mpl(s2, p['W3'], p['b3'])
    s2 = mpl(s2, p['W4'], p['b4'])
    s2 = jnp.concatenate([s1, s2], axis=1) @ p['W5'] + p['b5']
    s2 = bn(s2, p['g6'], p['be6'])
    s3 = green(s2, p['Wg3'], p['bg3'])
    s3 = bn(s3, p['g8'], p['be8'])
    s3 = green(s3, p['Wg4'], p['bg4'])
    s3 = mpl(s3, p['W10'], p['b10'])

    y11, st = _linear_multi([s3, s2], [p['W11'][:1024], p['W11'][1024:]],
                            p['b11'])
    s3p = _bn(y11, st, p['g12'], p['be12'], n)
    return _ffp([s3p, s2, s1],
                [p['Wf1'][:256], p['Wf1'][256:512], p['Wf1'][512:]],
                p['bf1'], p['Wf2'], p['bf2'])
